# Initial kernel scaffold; baseline (speedup 1.0000x reference)
import jax
import jax.numpy as jnp
from jax.experimental import pallas as pl


def kernel(x, edge_index, batch, W1, b1, W2, b2, LW1, Lb1, LW2, Lb2, LW3, Lb3):
    raise NotImplementedError("write your pallas kernel here")


# speedup vs baseline: 5.5807x; 5.5807x over previous
import functools

import jax
import jax.numpy as jnp
from jax import lax
from jax.experimental import pallas as pl
from jax.experimental.pallas import tpu as pltpu
from jax.experimental.pallas import tpu_sc as plsc

N = 10000
E = 320000
D = 128
H = 128
G = 64

NP = 10240
EP = 327680
NTILES = 32
EPT = EP // NTILES
CH = 128
ROWS_PT = NP // 16
BR = 1024
GP = 72

@functools.cache
def _sc_degree_kernel():
    return functools.partial(
        pl.kernel,
        out_type=jax.ShapeDtypeStruct((2, NP, D), jnp.float32),
        mesh=plsc.VectorSubcoreMesh(core_axis_name="c", subcore_axis_name="s"),
        scratch_types=[
            pltpu.VMEM((CH,), jnp.int32),
            pltpu.VMEM((CH, D), jnp.float32),
            pltpu.VMEM_SHARED((NP, D), jnp.float32),
        ],
    )(_sc_degree_body)


def _sc_degree_body(dst_hbm, ones_hbm, zeros_hbm, out_hbm, idx_v, ones_v, hist_sh):
    cid = lax.axis_index("c")
    sid = lax.axis_index("s")
    wid = cid * 16 + sid
    pltpu.sync_copy(zeros_hbm, hist_sh.at[pl.ds(sid * ROWS_PT, ROWS_PT)])
    pltpu.sync_copy(ones_hbm, ones_v)
    plsc.subcore_barrier()
    base = wid * EPT

    def body(g, carry):
        off = base + g * CH
        pltpu.sync_copy(dst_hbm.at[pl.ds(off, CH)], idx_v)
        pltpu.sync_copy(ones_v, hist_sh.at[idx_v], add=True)
        return carry

    lax.fori_loop(0, EPT // CH, body, 0)
    plsc.subcore_barrier()
    pltpu.sync_copy(
        hist_sh.at[pl.ds(sid * ROWS_PT, ROWS_PT)],
        out_hbm.at[cid, pl.ds(sid * ROWS_PT, ROWS_PT)],
    )


@functools.cache
def _sc_scatter_kernel():
    return functools.partial(
        pl.kernel,
        out_type=jax.ShapeDtypeStruct((2, NP, D), jnp.float32),
        mesh=plsc.VectorSubcoreMesh(core_axis_name="c", subcore_axis_name="s"),
        scratch_types=[
            pltpu.VMEM((CH,), jnp.int32),
            pltpu.VMEM((CH,), jnp.int32),
            pltpu.VMEM((CH, D), jnp.float32),
            pltpu.VMEM_SHARED((NP, D), jnp.float32),
            pltpu.SemaphoreType.DMA,
        ],
    )(_sc_scatter_body)


def _sc_scatter_body(hwp_hbm, src_hbm, dst_hbm, zeros_hbm, out_hbm,
                     src_v, dst_v, rows_v, acc_sh, sem):
    cid = lax.axis_index("c")
    sid = lax.axis_index("s")
    wid = cid * 16 + sid
    row0 = sid * ROWS_PT

    @pl.when(cid == 0)
    def _():
        pltpu.sync_copy(hwp_hbm.at[pl.ds(row0, ROWS_PT)],
                        acc_sh.at[pl.ds(row0, ROWS_PT)])

    @pl.when(cid != 0)
    def _():
        pltpu.sync_copy(zeros_hbm, acc_sh.at[pl.ds(row0, ROWS_PT)])

    plsc.subcore_barrier()
    base = wid * EPT

    def body(g, carry):
        off = base + g * CH
        pltpu.sync_copy(src_hbm.at[pl.ds(off, CH)], src_v)
        pltpu.sync_copy(dst_hbm.at[pl.ds(off, CH)], dst_v)
        pltpu.async_copy(hwp_hbm.at[src_v], rows_v, sem).wait()
        pltpu.sync_copy(rows_v, acc_sh.at[dst_v], add=True)
        return carry

    lax.fori_loop(0, EPT // CH, body, 0)
    plsc.subcore_barrier()
    pltpu.sync_copy(acc_sh.at[pl.ds(row0, ROWS_PT)],
                    out_hbm.at[cid, pl.ds(row0, ROWS_PT)])


def _dinv_block(d_ref):
    return d_ref[:, 0:1]


def _tc_first(xp, deg, W1):
    def body(x_ref, d_ref, w_ref, o_ref, dv_ref):
        degv = d_ref[0, :, 0:1] + d_ref[1, :, 0:1] + 1.0
        dinv = lax.rsqrt(degv)
        dv_ref[...] = dinv * jnp.ones((1, 16), jnp.float32)
        hw = jnp.dot(x_ref[...], w_ref[...], preferred_element_type=jnp.float32)
        o_ref[...] = dinv * hw

    return pl.pallas_call(
        body,
        grid=(NP // BR,),
        in_specs=[
            pl.BlockSpec((BR, D), lambda i: (i, 0)),
            pl.BlockSpec((2, BR, D), lambda i: (0, i, 0)),
            pl.BlockSpec((D, H), lambda i: (0, 0)),
        ],
        out_specs=[
            pl.BlockSpec((BR, H), lambda i: (i, 0)),
            pl.BlockSpec((BR, 16), lambda i: (i, 0)),
        ],
        out_shape=[
            jax.ShapeDtypeStruct((NP, H), jnp.float32),
            jax.ShapeDtypeStruct((NP, 16), jnp.float32),
        ],
    )(xp, deg, W1)


def _tc_mid(acc, dinv16, b_prev, W_next):
    def body(p_ref, d_ref, b_ref, w_ref, o_ref):
        dinv = _dinv_block(d_ref)
        a = p_ref[0] + p_ref[1]
        h = jnp.maximum(dinv * a + b_ref[...], 0.0)
        hw = jnp.dot(h, w_ref[...], preferred_element_type=jnp.float32)
        o_ref[...] = dinv * hw

    return pl.pallas_call(
        body,
        grid=(NP // BR,),
        in_specs=[
            pl.BlockSpec((2, BR, H), lambda i: (0, i, 0)),
            pl.BlockSpec((BR, 16), lambda i: (i, 0)),
            pl.BlockSpec((1, H), lambda i: (0, 0)),
            pl.BlockSpec((H, H), lambda i: (0, 0)),
        ],
        out_specs=pl.BlockSpec((BR, H), lambda i: (i, 0)),
        out_shape=jax.ShapeDtypeStruct((NP, H), jnp.float32),
    )(acc, dinv16, b_prev, W_next)


def _tc_pool_mlp(acc, dinv16, b_prev, batch2, LW1, Lb1, LW2, Lb2, LW3, Lb3):
    nblocks = NP // BR

    def body(p_ref, d_ref, b_ref, bat_ref, lw1, lb1, lw2, lb2, lw3, lb3,
             o_ref, pa, pm, pc):
        i = pl.program_id(0)

        @pl.when(i == 0)
        def _():
            pa[...] = jnp.zeros_like(pa)
            pc[...] = jnp.zeros_like(pc)
            pm[...] = jnp.full_like(pm, -jnp.inf)

        dinv = _dinv_block(d_ref)
        a = p_ref[0] + p_ref[1]
        h = jnp.maximum(dinv * a + b_ref[...], 0.0)

        bat = bat_ref[...]
        gids = lax.broadcasted_iota(jnp.int32, (1, GP), 1)
        mask = (bat == gids).astype(jnp.float32)
        pa[...] += lax.dot_general(mask, h, (((0,), (0,)), ((), ())),
                                   preferred_element_type=jnp.float32)
        pc[...] += lax.dot_general(mask, jnp.ones((BR, H), jnp.float32),
                                   (((0,), (0,)), ((), ())),
                                   preferred_element_type=jnp.float32)

        bmin = bat_ref[0, 0]
        bmax = bat_ref[BR - 1, 0]

        def mbody(g, carry):
            rowmask = bat == g
            vals = jnp.where(rowmask, h, -jnp.inf)
            m = jnp.max(vals, axis=0, keepdims=True)
            pm[pl.ds(g, 1), :] = jnp.maximum(pm[pl.ds(g, 1), :], m)
            return carry

        lax.fori_loop(bmin, bmax + 1, mbody, 0)

        @pl.when(i == nblocks - 1)
        def _():
            cnt = pc[0:G, :]
            padd = pa[0:G, :]
            mean = padd / jnp.maximum(cnt, 1.0)
            mx = jnp.where(cnt > 0.0, pm[0:G, :], 0.0)
            hcat = jnp.concatenate([mean, mx, padd], axis=1)
            y = jnp.dot(hcat, lw1[...], preferred_element_type=jnp.float32)
            y = jnp.maximum(y + lb1[...], 0.0)
            y = jnp.dot(y, lw2[...], preferred_element_type=jnp.float32)
            y = jnp.maximum(y + lb2[...], 0.0)
            y = jnp.dot(y, lw3[...], preferred_element_type=jnp.float32)
            y = y + lb3[...]
            o_ref[...] = 1.0 / (1.0 + jnp.exp(-y))

    return pl.pallas_call(
        body,
        grid=(nblocks,),
        in_specs=[
            pl.BlockSpec((2, BR, H), lambda i: (0, i, 0)),
            pl.BlockSpec((BR, 16), lambda i: (i, 0)),
            pl.BlockSpec((1, H), lambda i: (0, 0)),
            pl.BlockSpec((BR, 1), lambda i: (i, 0)),
            pl.BlockSpec((3 * H, 3 * H), lambda i: (0, 0)),
            pl.BlockSpec((1, 3 * H), lambda i: (0, 0)),
            pl.BlockSpec((3 * H, H), lambda i: (0, 0)),
            pl.BlockSpec((1, H), lambda i: (0, 0)),
            pl.BlockSpec((H, 1), lambda i: (0, 0)),
            pl.BlockSpec((1, 1), lambda i: (0, 0)),
        ],
        out_specs=pl.BlockSpec((G, 1), lambda i: (0, 0)),
        out_shape=jax.ShapeDtypeStruct((G, 1), jnp.float32),
        scratch_shapes=[
            pltpu.VMEM((GP, H), jnp.float32),
            pltpu.VMEM((GP, H), jnp.float32),
            pltpu.VMEM((GP, H), jnp.float32),
        ],
    )(acc, dinv16, b_prev, batch2, LW1, Lb1, LW2, Lb2, LW3, Lb3)


def kernel(x, edge_index, batch, W1, b1, W2, b2, LW1, Lb1, LW2, Lb2, LW3, Lb3):
    src = edge_index[0]
    dst = edge_index[1]
    pad_e = EP - E
    src_p = jnp.concatenate([src, jnp.zeros((pad_e,), jnp.int32)])
    dst_p = jnp.concatenate(
        [dst, N + (jnp.arange(pad_e, dtype=jnp.int32) % (NP - N))])
    xp = jnp.pad(x, ((0, NP - N), (0, 0)))
    batch2 = jnp.pad(batch, (0, NP - N), constant_values=G).reshape(NP, 1)
    onesD = jnp.ones((CH, D), jnp.float32)
    zerosD = jnp.zeros((ROWS_PT, D), jnp.float32)

    deg = _sc_degree_kernel()(dst_p, onesD, zerosD)

    hw1, dinv16 = _tc_first(xp, deg, W1)
    a1 = _sc_scatter_kernel()(hw1, src_p, dst_p, zerosD)
    hw2 = _tc_mid(a1, dinv16, b1.reshape(1, H), W2)
    a2 = _sc_scatter_kernel()(hw2, src_p, dst_p, zerosD)
    hw3 = _tc_mid(a2, dinv16, b2.reshape(1, H), W2)
    a3 = _sc_scatter_kernel()(hw3, src_p, dst_p, zerosD)

    return _tc_pool_mlp(a3, dinv16, b2.reshape(1, H), batch2,
                        LW1, Lb1.reshape(1, 3 * H), LW2, Lb2.reshape(1, H),
                        LW3, Lb3.reshape(1, 1))


# speedup vs baseline: 8.7037x; 1.5596x over previous
import functools

import jax
import jax.numpy as jnp
from jax import lax
from jax.experimental import pallas as pl
from jax.experimental.pallas import tpu as pltpu
from jax.experimental.pallas import tpu_sc as plsc

N = 10000
E = 320000
D = 128
H = 128
G = 64

NP = 10240
EP = 327680
NTILES = 32
EPT = EP // NTILES
CH = 128
NCH = EPT // CH
ROWS_PT = NP // 16
BR = 1024
GP = 72


def _zero_fill(buf):
    zvec = jnp.zeros((16,), jnp.float32)

    def zbody(r, carry):
        for c in range(D // 16):
            buf[r, pl.ds(c * 16, 16)] = zvec
        return carry

    lax.fori_loop(0, CH, zbody, 0)


@functools.cache
def _sc_degree_kernel():
    return functools.partial(
        pl.kernel,
        out_type=jax.ShapeDtypeStruct((2, NP, D), jnp.float32),
        mesh=plsc.VectorSubcoreMesh(core_axis_name="c", subcore_axis_name="s"),
        scratch_types=[
            pltpu.VMEM((NCH, CH), jnp.int32),
            pltpu.VMEM((CH, D), jnp.float32),
            pltpu.VMEM((CH, D), jnp.float32),
            pltpu.VMEM_SHARED((NP, D), jnp.float32),
            pltpu.SemaphoreType.DMA,
        ],
    )(_sc_degree_body)


def _sc_degree_body(dst_hbm, out_hbm, dsts_v, ones_v, zrow_v, hist_sh, dsem):
    cid = lax.axis_index("c")
    sid = lax.axis_index("s")
    wid = cid * 16 + sid
    row0 = sid * ROWS_PT

    ovec = jnp.ones((16,), jnp.float32)

    def fbody(r, carry):
        for c in range(D // 16):
            ones_v[r, pl.ds(c * 16, 16)] = ovec
        return carry

    lax.fori_loop(0, CH, fbody, 0)
    _zero_fill(zrow_v)
    for k in range(ROWS_PT // CH):
        pltpu.sync_copy(zrow_v, hist_sh.at[pl.ds(row0 + k * CH, CH)])
    pltpu.sync_copy(dst_hbm.at[wid], dsts_v)
    plsc.subcore_barrier()

    K = 8

    def round_body(r, carry):
        for j in range(K):
            pltpu.async_copy(ones_v, hist_sh.at[dsts_v.at[r * K + j]], dsem,
                             add=True)
        for j in range(K):
            pltpu.make_async_copy(ones_v, hist_sh.at[dsts_v.at[0]], dsem).wait()
        return carry

    lax.fori_loop(0, NCH // K, round_body, 0)
    plsc.subcore_barrier()
    pltpu.sync_copy(hist_sh.at[pl.ds(row0, ROWS_PT)],
                    out_hbm.at[cid, pl.ds(row0, ROWS_PT)])


@functools.cache
def _sc_scatter_kernel():
    return functools.partial(
        pl.kernel,
        out_type=jax.ShapeDtypeStruct((2, NP, D), jnp.float32),
        mesh=plsc.VectorSubcoreMesh(core_axis_name="c", subcore_axis_name="s"),
        scratch_types=[
            pltpu.VMEM((NCH // 2, CH), jnp.int32),
            pltpu.VMEM((NCH // 2, CH), jnp.int32),
            pltpu.VMEM((CH, D), jnp.float32),
            pltpu.VMEM((CH, D), jnp.float32),
            pltpu.VMEM_SHARED((NP, D), jnp.float32),
            pltpu.SemaphoreType.DMA,
            pltpu.SemaphoreType.DMA,
        ],
    )(_sc_scatter_body)


def _sc_scatter_body(hwp_hbm, src_hbm, dst_hbm, out_hbm,
                     srcs_v, dsts_v, rows0, rows1, acc_sh, gsem0, gsem1):
    cid = lax.axis_index("c")
    sid = lax.axis_index("s")
    wid = cid * 16 + sid
    row0 = sid * ROWS_PT

    _zero_fill(rows0)
    for k in range(ROWS_PT // CH):
        pltpu.sync_copy(rows0, acc_sh.at[pl.ds(row0 + k * CH, CH)])
    plsc.subcore_barrier()

    NCH2 = NCH // 2
    for half in range(2):
        pltpu.sync_copy(src_hbm.at[wid, pl.ds(half * NCH2, NCH2)], srcs_v)
        pltpu.sync_copy(dst_hbm.at[wid, pl.ds(half * NCH2, NCH2)], dsts_v)
        pltpu.async_copy(hwp_hbm.at[srcs_v.at[0]], rows0, gsem0)

        def pair_body(p, carry):
            c0 = 2 * p
            h1 = pltpu.async_copy(hwp_hbm.at[srcs_v.at[c0 + 1]], rows1, gsem1)
            pltpu.make_async_copy(hwp_hbm.at[srcs_v.at[0]], rows0, gsem0).wait()
            pltpu.sync_copy(rows0, acc_sh.at[dsts_v.at[c0]], add=True)
            nxt = jnp.where(c0 + 2 < NCH2, c0 + 2, 0)
            pltpu.async_copy(hwp_hbm.at[srcs_v.at[nxt]], rows0, gsem0)
            h1.wait()
            pltpu.sync_copy(rows1, acc_sh.at[dsts_v.at[c0 + 1]], add=True)
            return carry

        lax.fori_loop(0, NCH2 // 2, pair_body, 0)
        pltpu.make_async_copy(hwp_hbm.at[srcs_v.at[0]], rows0, gsem0).wait()
    plsc.subcore_barrier()
    pltpu.sync_copy(acc_sh.at[pl.ds(row0, ROWS_PT)],
                    out_hbm.at[cid, pl.ds(row0, ROWS_PT)])


def _tc_first(xp, deg, W1):
    def body(x_ref, d_ref, w_ref, o_ref, dv_ref):
        degv = d_ref[0, :, 0:1] + d_ref[1, :, 0:1] + 1.0
        dinv = lax.rsqrt(degv)
        dv_ref[...] = dinv * jnp.ones((1, 16), jnp.float32)
        hw = jnp.dot(x_ref[...], w_ref[...], preferred_element_type=jnp.float32)
        o_ref[...] = dinv * hw

    return pl.pallas_call(
        body,
        grid=(NP // BR,),
        in_specs=[
            pl.BlockSpec((BR, D), lambda i: (i, 0)),
            pl.BlockSpec((2, BR, D), lambda i: (0, i, 0)),
            pl.BlockSpec((D, H), lambda i: (0, 0)),
        ],
        out_specs=[
            pl.BlockSpec((BR, H), lambda i: (i, 0)),
            pl.BlockSpec((BR, 16), lambda i: (i, 0)),
        ],
        out_shape=[
            jax.ShapeDtypeStruct((NP, H), jnp.float32),
            jax.ShapeDtypeStruct((NP, 16), jnp.float32),
        ],
    )(xp, deg, W1)


def _tc_mid(acc, hwp, dinv16, b_prev, W_next):
    def body(p_ref, hw_ref, d_ref, b_ref, w_ref, o_ref):
        dinv = d_ref[:, 0:1]
        a = p_ref[0] + p_ref[1] + hw_ref[...]
        h = jnp.maximum(dinv * a + b_ref[...], 0.0)
        hw = jnp.dot(h, w_ref[...], preferred_element_type=jnp.float32)
        o_ref[...] = dinv * hw

    return pl.pallas_call(
        body,
        grid=(NP // BR,),
        in_specs=[
            pl.BlockSpec((2, BR, H), lambda i: (0, i, 0)),
            pl.BlockSpec((BR, H), lambda i: (i, 0)),
            pl.BlockSpec((BR, 16), lambda i: (i, 0)),
            pl.BlockSpec((1, H), lambda i: (0, 0)),
            pl.BlockSpec((H, H), lambda i: (0, 0)),
        ],
        out_specs=pl.BlockSpec((BR, H), lambda i: (i, 0)),
        out_shape=jax.ShapeDtypeStruct((NP, H), jnp.float32),
    )(acc, hwp, dinv16, b_prev, W_next)


def _tc_pool_mlp(acc, hwp, dinv16, b_prev, batch2, LW1, Lb1, LW2, Lb2, LW3, Lb3):
    nblocks = NP // BR

    def body(p_ref, hw_ref, d_ref, b_ref, bat_ref, lw1, lb1, lw2, lb2, lw3, lb3,
             o_ref, pa, pm, pc):
        i = pl.program_id(0)

        @pl.when(i == 0)
        def _():
            pa[...] = jnp.zeros_like(pa)
            pc[...] = jnp.zeros_like(pc)
            pm[...] = jnp.full_like(pm, -jnp.inf)

        dinv = d_ref[:, 0:1]
        a = p_ref[0] + p_ref[1] + hw_ref[...]
        h = jnp.maximum(dinv * a + b_ref[...], 0.0)

        bat = bat_ref[...]
        gids = lax.broadcasted_iota(jnp.int32, (1, GP), 1)
        mask = (bat == gids).astype(jnp.float32)
        pa[...] += lax.dot_general(mask, h, (((0,), (0,)), ((), ())),
                                   preferred_element_type=jnp.float32)
        pc[...] += lax.dot_general(mask, jnp.ones((BR, H), jnp.float32),
                                   (((0,), (0,)), ((), ())),
                                   preferred_element_type=jnp.float32)

        bmin = bat_ref[0, 0]
        bmax = bat_ref[BR - 1, 0]

        def mbody(g, carry):
            rowmask = bat == g
            vals = jnp.where(rowmask, h, -jnp.inf)
            m = jnp.max(vals, axis=0, keepdims=True)
            pm[pl.ds(g, 1), :] = jnp.maximum(pm[pl.ds(g, 1), :], m)
            return carry

        lax.fori_loop(bmin, bmax + 1, mbody, 0)

        @pl.when(i == nblocks - 1)
        def _():
            cnt = pc[0:G, :]
            padd = pa[0:G, :]
            mean = padd / jnp.maximum(cnt, 1.0)
            mx = jnp.where(cnt > 0.0, pm[0:G, :], 0.0)
            hcat = jnp.concatenate([mean, mx, padd], axis=1)
            y = jnp.dot(hcat, lw1[...], preferred_element_type=jnp.float32)
            y = jnp.maximum(y + lb1[...], 0.0)
            y = jnp.dot(y, lw2[...], preferred_element_type=jnp.float32)
            y = jnp.maximum(y + lb2[...], 0.0)
            y = jnp.dot(y, lw3[...], preferred_element_type=jnp.float32)
            y = y + lb3[...]
            o_ref[...] = 1.0 / (1.0 + jnp.exp(-y))

    return pl.pallas_call(
        body,
        grid=(nblocks,),
        in_specs=[
            pl.BlockSpec((2, BR, H), lambda i: (0, i, 0)),
            pl.BlockSpec((BR, H), lambda i: (i, 0)),
            pl.BlockSpec((BR, 16), lambda i: (i, 0)),
            pl.BlockSpec((1, H), lambda i: (0, 0)),
            pl.BlockSpec((BR, 1), lambda i: (i, 0)),
            pl.BlockSpec((3 * H, 3 * H), lambda i: (0, 0)),
            pl.BlockSpec((1, 3 * H), lambda i: (0, 0)),
            pl.BlockSpec((3 * H, H), lambda i: (0, 0)),
            pl.BlockSpec((1, H), lambda i: (0, 0)),
            pl.BlockSpec((H, 1), lambda i: (0, 0)),
            pl.BlockSpec((1, 1), lambda i: (0, 0)),
        ],
        out_specs=pl.BlockSpec((G, 1), lambda i: (0, 0)),
        out_shape=jax.ShapeDtypeStruct((G, 1), jnp.float32),
        scratch_shapes=[
            pltpu.VMEM((GP, H), jnp.float32),
            pltpu.VMEM((GP, H), jnp.float32),
            pltpu.VMEM((GP, H), jnp.float32),
        ],
    )(acc, hwp, dinv16, b_prev, batch2, LW1, Lb1, LW2, Lb2, LW3, Lb3)


def kernel(x, edge_index, batch, W1, b1, W2, b2, LW1, Lb1, LW2, Lb2, LW3, Lb3):
    src = edge_index[0]
    dst = edge_index[1]
    pad_e = EP - E
    src_p = jnp.concatenate([src, jnp.zeros((pad_e,), jnp.int32)])
    dst_p = jnp.concatenate(
        [dst, N + (jnp.arange(pad_e, dtype=jnp.int32) % (NP - N))])
    src_p3 = src_p.reshape(NTILES, NCH, CH)
    dst_p3 = dst_p.reshape(NTILES, NCH, CH)
    xp = jnp.pad(x, ((0, NP - N), (0, 0)))
    batch2 = jnp.pad(batch, (0, NP - N), constant_values=G).reshape(NP, 1)

    deg = _sc_degree_kernel()(dst_p3)

    hw1, dinv16 = _tc_first(xp, deg, W1)
    a1 = _sc_scatter_kernel()(hw1, src_p3, dst_p3)
    hw2 = _tc_mid(a1, hw1, dinv16, b1.reshape(1, H), W2)
    a2 = _sc_scatter_kernel()(hw2, src_p3, dst_p3)
    hw3 = _tc_mid(a2, hw2, dinv16, b2.reshape(1, H), W2)
    a3 = _sc_scatter_kernel()(hw3, src_p3, dst_p3)

    return _tc_pool_mlp(a3, hw3, dinv16, b2.reshape(1, H), batch2,
                        LW1, Lb1.reshape(1, 3 * H), LW2, Lb2.reshape(1, H),
                        LW3, Lb3.reshape(1, 1))
